# Initial kernel scaffold; baseline (speedup 1.0000x reference)
import jax
import jax.numpy as jnp
from jax.experimental import pallas as pl


def kernel(token_ids, base_embeddings):
    raise NotImplementedError("write your pallas kernel here")


# speedup vs baseline: 1.5402x; 1.5402x over previous
import functools

import jax
import jax.numpy as jnp
from jax import lax
from jax.experimental import pallas as pl
from jax.experimental.pallas import tpu as pltpu
from jax.experimental.pallas import tpu_sc as plsc

D = 1024
B = 8192
NC = 2
NS = 16
NW = NC * NS
B_PER_W = B // NW
C = 32
NCH = B_PER_W // C


def _make_sc_gather():
    mesh = plsc.VectorSubcoreMesh(core_axis_name="c", subcore_axis_name="s")

    @functools.partial(
        pl.kernel,
        mesh=mesh,
        out_type=jax.ShapeDtypeStruct((B, D), jnp.float32),
        scratch_types=[
            pltpu.VMEM((B_PER_W,), jnp.int32),
            pltpu.VMEM((C, D), jnp.float32),
            pltpu.VMEM((C, D), jnp.float32),
            pltpu.SemaphoreType.DMA,
            pltpu.SemaphoreType.DMA,
            pltpu.SemaphoreType.DMA,
            pltpu.SemaphoreType.DMA,
        ],
    )
    def gather_kernel(table_hbm, idx_hbm, out_hbm,
                      idx_v, rows0, rows1, g0, g1, w0, w1):
        wid = lax.axis_index("s") * NC + lax.axis_index("c")
        base = wid * B_PER_W
        pltpu.sync_copy(idx_hbm.at[pl.ds(base, B_PER_W)], idx_v)

        rows = (rows0, rows1)
        gsems = (g0, g1)
        wsems = (w0, w1)

        def start_gather(c):
            b = c % 2
            return pltpu.async_copy(
                table_hbm.at[idx_v.at[pl.ds(c * C, C)]], rows[b], gsems[b])

        def start_write(c):
            b = c % 2
            return pltpu.async_copy(
                rows[b], out_hbm.at[pl.ds(base + c * C, C)], wsems[b])

        g = [start_gather(0), start_gather(1)]
        w = [None, None]
        for c in range(NCH):
            b = c % 2
            g[b].wait()
            w[b] = start_write(c)
            if c + 2 < NCH:
                w[b].wait()
                g[b] = start_gather(c + 2)
        w[(NCH - 2) % 2].wait()
        w[(NCH - 1) % 2].wait()

    return gather_kernel


_sc_gather = _make_sc_gather()


@jax.jit
def kernel(token_ids, base_embeddings):
    tokens = token_ids.reshape(-1).astype(jnp.int32)
    out = _sc_gather(base_embeddings, tokens)
    return out.reshape(token_ids.shape + (base_embeddings.shape[-1],))


# speedup vs baseline: 1.5775x; 1.0242x over previous
import functools

import jax
import jax.numpy as jnp
from jax import lax
from jax.experimental import pallas as pl
from jax.experimental.pallas import tpu as pltpu
from jax.experimental.pallas import tpu_sc as plsc

D = 1024
B = 8192
NC = 2
NS = 16
NW = NC * NS
B_PER_W = B // NW
C = 32
NCH = B_PER_W // C


def _make_sc_gather():
    mesh = plsc.VectorSubcoreMesh(core_axis_name="c", subcore_axis_name="s")

    @functools.partial(
        pl.kernel,
        mesh=mesh,
        out_type=jax.ShapeDtypeStruct((B, D), jnp.float32),
        scratch_types=[
            pltpu.VMEM((B_PER_W,), jnp.int32),
            pltpu.VMEM((C, D), jnp.float32),
            pltpu.VMEM((C, D), jnp.float32),
            pltpu.VMEM((C, D), jnp.float32),
            pltpu.SemaphoreType.DMA,
            pltpu.SemaphoreType.DMA,
            pltpu.SemaphoreType.DMA,
            pltpu.SemaphoreType.DMA,
            pltpu.SemaphoreType.DMA,
            pltpu.SemaphoreType.DMA,
        ],
    )
    def gather_kernel(table_hbm, idx_hbm, out_hbm,
                      idx_v, rows0, rows1, rows2, g0, g1, g2, w0, w1, w2):
        wid = lax.axis_index("s") * NC + lax.axis_index("c")
        base = wid * B_PER_W
        pltpu.sync_copy(idx_hbm.at[pl.ds(base, B_PER_W)], idx_v)

        rows = (rows0, rows1, rows2)
        gsems = (g0, g1, g2)
        wsems = (w0, w1, w2)
        NB = 3

        def start_gather(c):
            b = c % NB
            return pltpu.async_copy(
                table_hbm.at[idx_v.at[pl.ds(c * C, C)]], rows[b], gsems[b])

        def start_write(c):
            b = c % NB
            return pltpu.async_copy(
                rows[b], out_hbm.at[pl.ds(base + c * C, C)], wsems[b])

        g = [start_gather(c) for c in range(NB)]
        w = [None] * NB
        for c in range(NCH):
            b = c % NB
            g[b].wait()
            w[b] = start_write(c)
            if c + NB < NCH:
                w[b].wait()
                g[b] = start_gather(c + NB)
        for c in range(NCH - min(NB, NCH), NCH):
            w[c % NB].wait()

    return gather_kernel


_sc_gather = _make_sc_gather()


@jax.jit
def kernel(token_ids, base_embeddings):
    tokens = token_ids.reshape(-1).astype(jnp.int32)
    out = _sc_gather(base_embeddings, tokens)
    return out.reshape(token_ids.shape + (base_embeddings.shape[-1],))


# speedup vs baseline: 1.5963x; 1.0119x over previous
import functools

import jax
import jax.numpy as jnp
from jax import lax
from jax.experimental import pallas as pl
from jax.experimental.pallas import tpu as pltpu
from jax.experimental.pallas import tpu_sc as plsc

D = 1024
B = 8192
NC = 2
NS = 16
NW = NC * NS
B_PER_W = B // NW
C = 16
NCH = B_PER_W // C


NB = 6


def _make_sc_gather():
    mesh = plsc.VectorSubcoreMesh(core_axis_name="c", subcore_axis_name="s")

    @functools.partial(
        pl.kernel,
        mesh=mesh,
        out_type=jax.ShapeDtypeStruct((B, D), jnp.float32),
        scratch_types=(
            [pltpu.VMEM((B_PER_W,), jnp.int32)]
            + [pltpu.VMEM((C, D), jnp.float32) for _ in range(NB)]
            + [pltpu.SemaphoreType.DMA for _ in range(2 * NB)]
        ),
    )
    def gather_kernel(table_hbm, idx_hbm, out_hbm, idx_v, *bufs):
        rows = bufs[:NB]
        gsems = bufs[NB:2 * NB]
        wsems = bufs[2 * NB:3 * NB]

        wid = lax.axis_index("s") * NC + lax.axis_index("c")
        base = wid * B_PER_W
        pltpu.sync_copy(idx_hbm.at[pl.ds(base, B_PER_W)], idx_v)

        def start_gather(c):
            b = c % NB
            return pltpu.async_copy(
                table_hbm.at[idx_v.at[pl.ds(c * C, C)]], rows[b], gsems[b])

        def start_write(c):
            b = c % NB
            return pltpu.async_copy(
                rows[b], out_hbm.at[pl.ds(base + c * C, C)], wsems[b])

        g = [start_gather(c) for c in range(min(NB, NCH))]
        w = [None] * NB
        for c in range(NCH):
            b = c % NB
            g[b].wait()
            w[b] = start_write(c)
            if c + NB < NCH:
                w[b].wait()
                g[b] = start_gather(c + NB)
        for c in range(max(0, NCH - NB), NCH):
            w[c % NB].wait()

    return gather_kernel


_sc_gather = _make_sc_gather()


@jax.jit
def kernel(token_ids, base_embeddings):
    tokens = token_ids.reshape(-1).astype(jnp.int32)
    out = _sc_gather(base_embeddings, tokens)
    return out.reshape(token_ids.shape + (base_embeddings.shape[-1],))
